# baseline (speedup 1.0000x reference)
import functools

import jax
import jax.numpy as jnp
from jax import lax
from jax.experimental import pallas as pl
from jax.experimental.pallas import tpu as pltpu
from jax.experimental.pallas import tpu_sc as plsc

HIDDEN = 2048
EXPERTS = 64
K = 8
BATCH = 4
SEQ = 4096
N_TOKENS = BATCH * SEQ
SEQ_BLOCK = 1024

_info = plsc.get_sparse_core_info()
NC, NS, LANES = _info.num_cores, _info.num_subcores, _info.num_lanes
NW = NC * NS
TOK_PER_W = N_TOKENS // NW
W_PER_B = SEQ // TOK_PER_W


def _matmul_body(x_ref, w_ref, logits_ref):
    logits_ref[0] = lax.dot_general(
        w_ref[...], x_ref[0], (((1,), (1,)), ((), ())),
        preferred_element_type=jnp.float32,
    )


def _topk_body(logits_hbm, wts_hbm, idx_hbm, logits_v, wts_v, idx_v):
    wid = lax.axis_index("s") * NC + lax.axis_index("c")
    b = wid // W_PER_B
    s0 = (wid % W_PER_B) * TOK_PER_W
    pltpu.sync_copy(logits_hbm.at[b, :, pl.ds(s0, TOK_PER_W)], logits_v)

    lane = jnp.arange(LANES, dtype=jnp.int32)
    mask8 = lane < K
    idx_consts = [lane + 16 * c for c in range(4)]

    def merge(ak, ai, bk, bi):
        rbk = lax.rev(bk, (0,))
        rbi = lax.rev(bi, (0,))
        ge = ak >= rbk
        hk = jnp.where(ge, ak, rbk)
        hi = jnp.where(ge, ai, rbi)
        return plsc.sort_key_val(hk, hi, descending=True)

    @plsc.parallel_loop(0, TOK_PER_W, step=1, unroll=8)
    def body(t):
        tvec = jnp.full((LANES,), t, dtype=jnp.int32)
        leafs = [
            plsc.sort_key_val(plsc.load_gather(logits_v, [idx_consts[c], tvec]),
                              idx_consts[c], descending=True)
            for c in range(4)
        ]
        k01, i01 = merge(*leafs[0], *leafs[1])
        k23, i23 = merge(*leafs[2], *leafs[3])
        fk, fi = merge(k01, i01, k23, i23)
        e = jnp.where(mask8, jnp.exp(fk - jnp.max(fk)), 0.0)
        w8 = e / jnp.sum(e)
        plsc.store_scatter(wts_v, [lane, tvec], w8, mask=mask8)
        plsc.store_scatter(idx_v, [lane, tvec], fi, mask=mask8)

    pltpu.sync_copy(wts_v, wts_hbm.at[b, :, pl.ds(s0, TOK_PER_W)])
    pltpu.sync_copy(idx_v, idx_hbm.at[b, :, pl.ds(s0, TOK_PER_W)])


_topk_call = pl.kernel(
    _topk_body,
    out_type=[
        jax.ShapeDtypeStruct((BATCH, K, SEQ), jnp.float32),
        jax.ShapeDtypeStruct((BATCH, K, SEQ), jnp.int32),
    ],
    mesh=plsc.VectorSubcoreMesh(core_axis_name="c", subcore_axis_name="s"),
    compiler_params=pltpu.CompilerParams(needs_layout_passes=False),
    scratch_types=[
        pltpu.VMEM((EXPERTS, TOK_PER_W), jnp.float32),
        pltpu.VMEM((K, TOK_PER_W), jnp.float32),
        pltpu.VMEM((K, TOK_PER_W), jnp.int32),
    ],
)


@functools.partial(jax.jit, static_argnames=())
def kernel(hidden_states, gate_weight):
    B, S, H = hidden_states.shape
    logits_bt = pl.pallas_call(
        _matmul_body,
        grid=(B, S // SEQ_BLOCK),
        in_specs=[
            pl.BlockSpec((1, SEQ_BLOCK, H), lambda b, j: (b, j, 0)),
            pl.BlockSpec((EXPERTS, H), lambda b, j: (0, 0)),
        ],
        out_specs=pl.BlockSpec((1, EXPERTS, SEQ_BLOCK), lambda b, j: (b, 0, j)),
        out_shape=jax.ShapeDtypeStruct((B, EXPERTS, S), jnp.float32),
    )(hidden_states, gate_weight)
    wts_p, idx_p = _topk_call(logits_bt)
    return (jnp.swapaxes(logits_bt, 1, 2),
            jnp.swapaxes(wts_p, 1, 2),
            jnp.swapaxes(idx_p, 1, 2))


# speedup vs baseline: 1.0449x; 1.0449x over previous
import functools

import jax
import jax.numpy as jnp
from jax import lax
from jax.experimental import pallas as pl
from jax.experimental.pallas import tpu as pltpu
from jax.experimental.pallas import tpu_sc as plsc

HIDDEN = 2048
EXPERTS = 64
K = 8
BATCH = 4
SEQ = 4096
SEQ_BLOCK = 1024
HALF_B = BATCH // 2

_info = plsc.get_sparse_core_info()
NC, NS, LANES = _info.num_cores, _info.num_subcores, _info.num_lanes
NW = NC * NS
TOK_PER_W = HALF_B * SEQ // NW
W_PER_B = SEQ // TOK_PER_W


def _matmul_body(x_ref, w_ref, logits_ref):
    logits_ref[0] = lax.dot_general(
        w_ref[...], x_ref[0], (((1,), (1,)), ((), ())),
        preferred_element_type=jnp.float32,
    )


def _topk_body(logits_hbm, wts_hbm, idx_hbm, logits_v, wts_v, idx_v):
    wid = lax.axis_index("s") * NC + lax.axis_index("c")
    b = wid // W_PER_B
    s0 = (wid % W_PER_B) * TOK_PER_W
    pltpu.sync_copy(logits_hbm.at[b, :, pl.ds(s0, TOK_PER_W)], logits_v)

    lane = jnp.arange(LANES, dtype=jnp.int32)
    mask8 = lane < K
    idx_consts = [lane + 16 * c for c in range(4)]

    def merge(ak, ai, bk, bi):
        rbk = lax.rev(bk, (0,))
        rbi = lax.rev(bi, (0,))
        ge = ak >= rbk
        hk = jnp.where(ge, ak, rbk)
        hi = jnp.where(ge, ai, rbi)
        return plsc.sort_key_val(hk, hi, descending=True)

    @plsc.parallel_loop(0, TOK_PER_W, step=1, unroll=4)
    def body(t):
        tvec = jnp.full((LANES,), t, dtype=jnp.int32)
        leafs = [
            plsc.sort_key_val(plsc.load_gather(logits_v, [idx_consts[c], tvec]),
                              idx_consts[c], descending=True)
            for c in range(4)
        ]
        k01, i01 = merge(*leafs[0], *leafs[1])
        k23, i23 = merge(*leafs[2], *leafs[3])
        fk, fi = merge(k01, i01, k23, i23)
        e = jnp.where(mask8, jnp.exp(fk - jnp.max(fk)), 0.0)
        w8 = e / jnp.sum(e)
        plsc.store_scatter(wts_v, [lane, tvec], w8, mask=mask8)
        plsc.store_scatter(idx_v, [lane, tvec], fi, mask=mask8)

    pltpu.sync_copy(wts_v, wts_hbm.at[b, :, pl.ds(s0, TOK_PER_W)])
    pltpu.sync_copy(idx_v, idx_hbm.at[b, :, pl.ds(s0, TOK_PER_W)])


_topk_call = pl.kernel(
    _topk_body,
    out_type=[
        jax.ShapeDtypeStruct((HALF_B, K, SEQ), jnp.float32),
        jax.ShapeDtypeStruct((HALF_B, K, SEQ), jnp.int32),
    ],
    mesh=plsc.VectorSubcoreMesh(core_axis_name="c", subcore_axis_name="s"),
    compiler_params=pltpu.CompilerParams(needs_layout_passes=False),
    scratch_types=[
        pltpu.VMEM((EXPERTS, TOK_PER_W), jnp.float32),
        pltpu.VMEM((K, TOK_PER_W), jnp.float32),
        pltpu.VMEM((K, TOK_PER_W), jnp.int32),
    ],
)


def _half_matmul(hidden_states, gate_weight, half):
    return pl.pallas_call(
        _matmul_body,
        grid=(HALF_B, SEQ // SEQ_BLOCK),
        in_specs=[
            pl.BlockSpec((1, SEQ_BLOCK, HIDDEN),
                         lambda b, j: (b + half * HALF_B, j, 0)),
            pl.BlockSpec((EXPERTS, HIDDEN), lambda b, j: (0, 0)),
        ],
        out_specs=pl.BlockSpec((1, EXPERTS, SEQ_BLOCK), lambda b, j: (b, 0, j)),
        out_shape=jax.ShapeDtypeStruct((HALF_B, EXPERTS, SEQ), jnp.float32),
    )(hidden_states, gate_weight)


@functools.partial(jax.jit, static_argnames=())
def kernel(hidden_states, gate_weight):
    B, S, H = hidden_states.shape
    log_a = _half_matmul(hidden_states, gate_weight, 0)
    log_b = _half_matmul(hidden_states, gate_weight, 1)
    wts_a, idx_a = _topk_call(log_a)
    wts_b, idx_b = _topk_call(log_b)
    logits = jnp.concatenate([log_a, log_b], axis=0)
    wts_p = jnp.concatenate([wts_a, wts_b], axis=0)
    idx_p = jnp.concatenate([idx_a, idx_b], axis=0)
    return (jnp.swapaxes(logits, 1, 2),
            jnp.swapaxes(wts_p, 1, 2),
            jnp.swapaxes(idx_p, 1, 2))
